# baseline (speedup 1.0000x reference)
import functools

import jax
import jax.numpy as jnp
from jax import lax
from jax.experimental import pallas as pl
from jax.experimental.pallas import tpu as pltpu
from jax.experimental.pallas import tpu_sc as plsc

N = 10000
E = 320000
C = 128
HID = 256

NC = 2
NS = 16
NW = NC * NS
CK = 128
CPT = 160
CPT0 = 157
CPT1 = 3
GN = 20
NG = CPT // GN
EPT = 10240
EPAD = EPT * NW
NPAD = 10112
RPT = NPAD // NS

_mesh = plsc.VectorSubcoreMesh(core_axis_name="c", subcore_axis_name="s")
_sc_params = pltpu.CompilerParams(needs_layout_passes=False)


@functools.partial(
    pl.kernel,
    out_type=jax.ShapeDtypeStruct((NW, NPAD), jnp.float32),
    mesh=_mesh,
    compiler_params=_sc_params,
    scratch_types=[
        pltpu.VMEM((EPT,), jnp.int32),
        pltpu.VMEM((NPAD,), jnp.float32),
    ],
)
def _sc_degree(dst_hbm, out_hbm, didx, cnt):
    c = lax.axis_index("c")
    s = lax.axis_index("s")
    wid = c * NS + s

    def zero(i, _):
        cnt[pl.ds(i * 16, 16)] = jnp.zeros((16,), jnp.float32)
        return 0

    lax.fori_loop(0, NPAD // 16, zero, 0)
    pltpu.sync_copy(dst_hbm.at[wid], didx)
    ones = jnp.ones((16,), jnp.float32)

    def body(i, _):
        idx = didx[pl.ds(i * 16, 16)]
        plsc.addupdate_scatter(cnt, [idx], ones)
        return 0

    lax.fori_loop(0, EPT // 16, body, 0)
    pltpu.sync_copy(cnt, out_hbm.at[wid])


@functools.partial(
    pl.kernel,
    out_type=jax.ShapeDtypeStruct((NC, NPAD, C), jnp.float32),
    mesh=_mesh,
    compiler_params=_sc_params,
    scratch_types=[
        pltpu.VMEM((GN, 2, CK), jnp.int32),
        pltpu.VMEM((GN, 2, CK), jnp.int32),
        pltpu.VMEM((CK, C), jnp.float32),
        pltpu.VMEM((CK, C), jnp.float32),
        pltpu.VMEM_SHARED((NPAD, C), jnp.float32),
        pltpu.SemaphoreType.DMA,
        pltpu.SemaphoreType.DMA,
        pltpu.SemaphoreType.DMA,
    ],
)
def _sc_aggregate(v_hbm, idx_hbm, out_hbm, ib0, ib1, rb0, rb1, acc,
                  sg0, sg1, si):
    c = lax.axis_index("c")
    s = lax.axis_index("s")
    wid = c * NS + s

    def zrow(r, _):
        def zlane(l, _):
            rb0[r, pl.ds(l * 16, 16)] = jnp.zeros((16,), jnp.float32)
            return 0
        return lax.fori_loop(0, C // 16, zlane, 0)

    lax.fori_loop(0, CK, zrow, 0)

    base = s * RPT
    off = 0
    for step in (CK,) * (RPT // CK) + (RPT % CK,):
        pltpu.sync_copy(rb0.at[pl.ds(0, step)], acc.at[pl.ds(base + off, step)])
        off += step

    pltpu.sync_copy(idx_hbm.at[wid, pl.ds(0, GN)], ib0)
    plsc.subcore_barrier()

    my_cpt = jnp.where(c == 0, CPT0, CPT1)

    def start(ib, j, rb, sem):
        pltpu.async_copy(v_hbm.at[ib.at[j, 0]], rb, sem)

    def drain(rb, sem):
        pltpu.make_async_copy(v_hbm.at[pl.ds(0, CK)], rb, sem).wait()

    def scat(ib, j, rb):
        pltpu.sync_copy(rb, acc.at[ib.at[j, 1]], add=True)

    def when_real(gj, fn):
        pl.when(gj < my_cpt)(fn)

    start(ib0, 0, rb0, sg0)
    ibufs = (ib0, ib1)
    for g in range(NG):
        ib = ibufs[g % 2]
        nxt = ibufs[(g + 1) % 2]
        G = g * GN
        if g + 1 < NG:
            pltpu.async_copy(idx_hbm.at[wid, pl.ds((g + 1) * GN, GN)], nxt, si)

        def body(i, _, ib=ib, G=G):
            j = 2 * i
            when_real(G + j + 1, lambda: start(ib, j + 1, rb1, sg1))

            def d0():
                drain(rb0, sg0)
                scat(ib, j, rb0)
            when_real(G + j, d0)
            when_real(G + j + 2, lambda: start(ib, j + 2, rb0, sg0))

            def d1():
                drain(rb1, sg1)
                scat(ib, j + 1, rb1)
            when_real(G + j + 1, d1)
            return 0

        lax.fori_loop(0, GN // 2 - 1, body, 0)
        when_real(G + GN - 1, lambda ib=ib: start(ib, GN - 1, rb1, sg1))

        def dl0(ib=ib, G=G):
            drain(rb0, sg0)
            scat(ib, GN - 2, rb0)
        when_real(G + GN - 2, dl0)
        if g + 1 < NG:
            pltpu.make_async_copy(
                idx_hbm.at[wid, pl.ds(0, GN)], nxt, si).wait()
            when_real(G + GN, lambda nxt=nxt: start(nxt, 0, rb0, sg0))

        def dl1(ib=ib, G=G):
            drain(rb1, sg1)
            scat(ib, GN - 1, rb1)
        when_real(G + GN - 1, dl1)

    plsc.subcore_barrier()
    sl = pl.ds(base, RPT)
    pltpu.sync_copy(acc.at[sl], out_hbm.at[c, sl])


_BN = 2000


def _tc_dinv_body(degp_ref, dinvb_ref):
    deg = jnp.sum(degp_ref[...], axis=0) + 1.0
    dinv = lax.rsqrt(deg)
    dinvb_ref[...] = jnp.broadcast_to(dinv[:, None], (NPAD, C))


def _tc_dinv(degp):
    return pl.pallas_call(
        _tc_dinv_body,
        out_shape=jax.ShapeDtypeStruct((NPAD, C), jnp.float32),
    )(degp)


def _tc_prep_body(dinvb_ref, x_ref, v1_ref):
    v1_ref[...] = x_ref[...] * dinvb_ref[...]


def _tc_prep(dinvb, x):
    return pl.pallas_call(
        _tc_prep_body,
        grid=(N // _BN,),
        in_specs=[
            pl.BlockSpec((_BN, C), lambda i: (i, 0)),
            pl.BlockSpec((_BN, C), lambda i: (i, 0)),
        ],
        out_specs=pl.BlockSpec((_BN, C), lambda i: (i, 0)),
        out_shape=jax.ShapeDtypeStruct((N, C), jnp.float32),
    )(dinvb, x)


def _tc_mid_body(aggp_ref, v1_ref, dinvb_ref, W1_ref, b1_ref, W2_ref, v2_ref):
    dinvb = dinvb_ref[...]
    pre = (aggp_ref[0] + aggp_ref[1] + v1_ref[...]) * dinvb
    h = jnp.dot(pre, W1_ref[...], preferred_element_type=jnp.float32)
    h = jnp.maximum(h + b1_ref[...], 0.0)
    v2_ref[...] = jnp.dot(h, W2_ref[...],
                          preferred_element_type=jnp.float32) * dinvb


def _tc_mid(aggp, v1, dinvb, W1, b1, W2):
    return pl.pallas_call(
        _tc_mid_body,
        grid=(N // _BN,),
        in_specs=[
            pl.BlockSpec((NC, _BN, C), lambda i: (0, i, 0)),
            pl.BlockSpec((_BN, C), lambda i: (i, 0)),
            pl.BlockSpec((_BN, C), lambda i: (i, 0)),
            pl.BlockSpec((C, HID), lambda i: (0, 0)),
            pl.BlockSpec((1, HID), lambda i: (0, 0)),
            pl.BlockSpec((HID, C), lambda i: (0, 0)),
        ],
        out_specs=pl.BlockSpec((_BN, C), lambda i: (i, 0)),
        out_shape=jax.ShapeDtypeStruct((N, C), jnp.float32),
    )(aggp, v1, dinvb, W1, b1, W2)


def _tc_final_body(aggp_ref, v2_ref, dinvb_ref, b2_ref, out_ref):
    out_ref[...] = (aggp_ref[0] + aggp_ref[1] + v2_ref[...]) * dinvb_ref[...] \
        + b2_ref[...]


def _tc_final(aggp, v2, dinvb, b2):
    return pl.pallas_call(
        _tc_final_body,
        grid=(N // _BN,),
        in_specs=[
            pl.BlockSpec((NC, _BN, C), lambda i: (0, i, 0)),
            pl.BlockSpec((_BN, C), lambda i: (i, 0)),
            pl.BlockSpec((_BN, C), lambda i: (i, 0)),
            pl.BlockSpec((1, C), lambda i: (0, 0)),
        ],
        out_specs=pl.BlockSpec((_BN, C), lambda i: (i, 0)),
        out_shape=jax.ShapeDtypeStruct((N, C), jnp.float32),
    )(aggp, v2, dinvb, b2)


def kernel(x, edge_index, W1, b1, W2, b2):
    src = edge_index[0].astype(jnp.int32)
    dst = edge_index[1].astype(jnp.int32)
    pad = EPAD - E
    src_p = jnp.concatenate([src, jnp.zeros((pad,), jnp.int32)])
    dst_p = jnp.concatenate([dst, jnp.full((pad,), N, jnp.int32)])

    n0 = NS * CPT0 * CK

    def build(a):
        a0 = a[:n0].reshape(NS, CPT0, CK)
        a1 = a[n0:].reshape(NS, CPT1, CK)
        z0 = jnp.zeros((NS, CPT - CPT0, CK), jnp.int32)
        z1 = jnp.zeros((NS, CPT - CPT1, CK), jnp.int32)
        return jnp.concatenate([
            jnp.concatenate([a0, z0], axis=1),
            jnp.concatenate([a1, z1], axis=1),
        ], axis=0)

    idx = jnp.stack([build(src_p), build(dst_p)], axis=2)
    dst2 = dst_p.reshape(NW, EPT)

    degp = _sc_degree(dst2)
    dinvb = _tc_dinv(degp)[:N]
    v1 = _tc_prep(dinvb, x)
    agg1 = _sc_aggregate(v1, idx)
    v2 = _tc_mid(agg1[:, :N, :], v1, dinvb, W1, b1.reshape(1, HID), W2)
    agg2 = _sc_aggregate(v2, idx)
    out = _tc_final(agg2[:, :N, :], v2, dinvb, b2.reshape(1, C))
    return out


# speedup vs baseline: 1.0910x; 1.0910x over previous
import functools

import jax
import jax.numpy as jnp
from jax import lax
from jax.experimental import pallas as pl
from jax.experimental.pallas import tpu as pltpu
from jax.experimental.pallas import tpu_sc as plsc

N = 10000
E = 320000
C = 128
HID = 256

NC = 2
NS = 16
NW = NC * NS
CK = 128
CPT = 160
CPT0 = 150
CPT1 = 10
GN = 20
NG = CPT // GN
EPT = 10240
EPAD = EPT * NW
NPAD = 10112
RPT = NPAD // NS

_mesh = plsc.VectorSubcoreMesh(core_axis_name="c", subcore_axis_name="s")
_sc_params = pltpu.CompilerParams(needs_layout_passes=False)


@functools.partial(
    pl.kernel,
    out_type=jax.ShapeDtypeStruct((NW, NPAD), jnp.float32),
    mesh=_mesh,
    compiler_params=_sc_params,
    scratch_types=[
        pltpu.VMEM((EPT,), jnp.int32),
        pltpu.VMEM((NPAD,), jnp.float32),
    ],
)
def _sc_degree(dst_hbm, out_hbm, didx, cnt):
    c = lax.axis_index("c")
    s = lax.axis_index("s")
    wid = c * NS + s

    def zero(i, _):
        cnt[pl.ds(i * 16, 16)] = jnp.zeros((16,), jnp.float32)
        return 0

    lax.fori_loop(0, NPAD // 16, zero, 0)
    pltpu.sync_copy(dst_hbm.at[wid], didx)
    ones = jnp.ones((16,), jnp.float32)

    def body(i, _):
        idx = didx[pl.ds(i * 16, 16)]
        plsc.addupdate_scatter(cnt, [idx], ones)
        return 0

    lax.fori_loop(0, EPT // 16, body, 0)
    pltpu.sync_copy(cnt, out_hbm.at[wid])


@functools.partial(
    pl.kernel,
    out_type=jax.ShapeDtypeStruct((NC, NPAD, C), jnp.float32),
    mesh=_mesh,
    compiler_params=_sc_params,
    scratch_types=[
        pltpu.VMEM((GN, 2, CK), jnp.int32),
        pltpu.VMEM((GN, 2, CK), jnp.int32),
        pltpu.VMEM((CK, C), jnp.float32),
        pltpu.VMEM((CK, C), jnp.float32),
        pltpu.VMEM_SHARED((NPAD, C), jnp.float32),
        pltpu.SemaphoreType.DMA,
        pltpu.SemaphoreType.DMA,
        pltpu.SemaphoreType.DMA,
    ],
)
def _sc_aggregate(v_hbm, idx_hbm, out_hbm, ib0, ib1, rb0, rb1, acc,
                  sg0, sg1, si):
    c = lax.axis_index("c")
    s = lax.axis_index("s")
    wid = c * NS + s

    def zrow(r, _):
        def zlane(l, _):
            rb0[r, pl.ds(l * 16, 16)] = jnp.zeros((16,), jnp.float32)
            return 0
        return lax.fori_loop(0, C // 16, zlane, 0)

    lax.fori_loop(0, CK, zrow, 0)

    base = s * RPT
    off = 0
    for step in (CK,) * (RPT // CK) + (RPT % CK,):
        pltpu.sync_copy(rb0.at[pl.ds(0, step)], acc.at[pl.ds(base + off, step)])
        off += step

    pltpu.sync_copy(idx_hbm.at[wid, pl.ds(0, GN)], ib0)
    plsc.subcore_barrier()

    my_cpt = jnp.where(c == 0, CPT0, CPT1)

    def start(ib, j, rb, sem):
        pltpu.async_copy(v_hbm.at[ib.at[j, 0]], rb, sem)

    def drain(rb, sem):
        pltpu.make_async_copy(v_hbm.at[pl.ds(0, CK)], rb, sem).wait()

    def scat(ib, j, rb):
        pltpu.sync_copy(rb, acc.at[ib.at[j, 1]], add=True)

    def when_real(gj, fn):
        pl.when(gj < my_cpt)(fn)

    start(ib0, 0, rb0, sg0)
    ibufs = (ib0, ib1)
    for g in range(NG):
        ib = ibufs[g % 2]
        nxt = ibufs[(g + 1) % 2]
        G = g * GN
        if g + 1 < NG:
            pltpu.async_copy(idx_hbm.at[wid, pl.ds((g + 1) * GN, GN)], nxt, si)

        def body(i, _, ib=ib, G=G):
            j = 2 * i
            when_real(G + j + 1, lambda: start(ib, j + 1, rb1, sg1))

            def d0():
                drain(rb0, sg0)
                scat(ib, j, rb0)
            when_real(G + j, d0)
            when_real(G + j + 2, lambda: start(ib, j + 2, rb0, sg0))

            def d1():
                drain(rb1, sg1)
                scat(ib, j + 1, rb1)
            when_real(G + j + 1, d1)
            return 0

        lax.fori_loop(0, GN // 2 - 1, body, 0)
        when_real(G + GN - 1, lambda ib=ib: start(ib, GN - 1, rb1, sg1))

        def dl0(ib=ib, G=G):
            drain(rb0, sg0)
            scat(ib, GN - 2, rb0)
        when_real(G + GN - 2, dl0)
        if g + 1 < NG:
            pltpu.make_async_copy(
                idx_hbm.at[wid, pl.ds(0, GN)], nxt, si).wait()
            when_real(G + GN, lambda nxt=nxt: start(nxt, 0, rb0, sg0))

        def dl1(ib=ib, G=G):
            drain(rb1, sg1)
            scat(ib, GN - 1, rb1)
        when_real(G + GN - 1, dl1)

    plsc.subcore_barrier()
    sl = pl.ds(base, RPT)
    pltpu.sync_copy(acc.at[sl], out_hbm.at[c, sl])


_BN = 2000


def _tc_dinv_body(degp_ref, dinvb_ref):
    deg = jnp.sum(degp_ref[...], axis=0) + 1.0
    dinv = lax.rsqrt(deg)
    dinvb_ref[...] = jnp.broadcast_to(dinv[:, None], (NPAD, C))


def _tc_dinv(degp):
    return pl.pallas_call(
        _tc_dinv_body,
        out_shape=jax.ShapeDtypeStruct((NPAD, C), jnp.float32),
    )(degp)


def _tc_prep_body(dinvb_ref, x_ref, v1_ref):
    v1_ref[...] = x_ref[...] * dinvb_ref[...]


def _tc_prep(dinvb, x):
    return pl.pallas_call(
        _tc_prep_body,
        grid=(N // _BN,),
        in_specs=[
            pl.BlockSpec((_BN, C), lambda i: (i, 0)),
            pl.BlockSpec((_BN, C), lambda i: (i, 0)),
        ],
        out_specs=pl.BlockSpec((_BN, C), lambda i: (i, 0)),
        out_shape=jax.ShapeDtypeStruct((N, C), jnp.float32),
    )(dinvb, x)


def _tc_mid_body(aggp_ref, v1_ref, dinvb_ref, W1_ref, b1_ref, W2_ref, v2_ref):
    dinvb = dinvb_ref[...]
    pre = (aggp_ref[0] + aggp_ref[1] + v1_ref[...]) * dinvb
    h = jnp.dot(pre, W1_ref[...], preferred_element_type=jnp.float32)
    h = jnp.maximum(h + b1_ref[...], 0.0)
    v2_ref[...] = jnp.dot(h, W2_ref[...],
                          preferred_element_type=jnp.float32) * dinvb


def _tc_mid(aggp, v1, dinvb, W1, b1, W2):
    return pl.pallas_call(
        _tc_mid_body,
        grid=(N // _BN,),
        in_specs=[
            pl.BlockSpec((NC, _BN, C), lambda i: (0, i, 0)),
            pl.BlockSpec((_BN, C), lambda i: (i, 0)),
            pl.BlockSpec((_BN, C), lambda i: (i, 0)),
            pl.BlockSpec((C, HID), lambda i: (0, 0)),
            pl.BlockSpec((1, HID), lambda i: (0, 0)),
            pl.BlockSpec((HID, C), lambda i: (0, 0)),
        ],
        out_specs=pl.BlockSpec((_BN, C), lambda i: (i, 0)),
        out_shape=jax.ShapeDtypeStruct((N, C), jnp.float32),
    )(aggp, v1, dinvb, W1, b1, W2)


def _tc_final_body(aggp_ref, v2_ref, dinvb_ref, b2_ref, out_ref):
    out_ref[...] = (aggp_ref[0] + aggp_ref[1] + v2_ref[...]) * dinvb_ref[...] \
        + b2_ref[...]


def _tc_final(aggp, v2, dinvb, b2):
    return pl.pallas_call(
        _tc_final_body,
        grid=(N // _BN,),
        in_specs=[
            pl.BlockSpec((NC, _BN, C), lambda i: (0, i, 0)),
            pl.BlockSpec((_BN, C), lambda i: (i, 0)),
            pl.BlockSpec((_BN, C), lambda i: (i, 0)),
            pl.BlockSpec((1, C), lambda i: (0, 0)),
        ],
        out_specs=pl.BlockSpec((_BN, C), lambda i: (i, 0)),
        out_shape=jax.ShapeDtypeStruct((N, C), jnp.float32),
    )(aggp, v2, dinvb, b2)


def kernel(x, edge_index, W1, b1, W2, b2):
    src = edge_index[0].astype(jnp.int32)
    dst = edge_index[1].astype(jnp.int32)
    pad = EPAD - E
    src_p = jnp.concatenate([src, jnp.zeros((pad,), jnp.int32)])
    dst_p = jnp.concatenate([dst, jnp.full((pad,), N, jnp.int32)])

    n0 = NS * CPT0 * CK

    def build(a):
        a0 = a[:n0].reshape(NS, CPT0, CK)
        a1 = a[n0:].reshape(NS, CPT1, CK)
        z0 = jnp.zeros((NS, CPT - CPT0, CK), jnp.int32)
        z1 = jnp.zeros((NS, CPT - CPT1, CK), jnp.int32)
        return jnp.concatenate([
            jnp.concatenate([a0, z0], axis=1),
            jnp.concatenate([a1, z1], axis=1),
        ], axis=0)

    idx = jnp.stack([build(src_p), build(dst_p)], axis=2)
    dst2 = dst_p.reshape(NW, EPT)

    degp = _sc_degree(dst2)
    dinvb = _tc_dinv(degp)[:N]
    v1 = _tc_prep(dinvb, x)
    agg1 = _sc_aggregate(v1, idx)
    v2 = _tc_mid(agg1[:, :N, :], v1, dinvb, W1, b1.reshape(1, HID), W2)
    agg2 = _sc_aggregate(v2, idx)
    out = _tc_final(agg2[:, :N, :], v2, dinvb, b2.reshape(1, C))
    return out
